# baseline (speedup 1.0000x reference)
import functools

import jax
import jax.numpy as jnp
from jax import lax
from jax.experimental import pallas as pl
from jax.experimental.pallas import tpu as pltpu
from jax.experimental.pallas import tpu_sc as plsc

B = 4096
H = 50
NC, NS = 2, 16
NW = NC * NS
BW = B // NW
CY = 64
NYC = B // NW // CY

_f32 = jnp.float32
_i32 = jnp.int32


def _make_kernel():
    mesh = plsc.VectorSubcoreMesh(
        core_axis_name="c", subcore_axis_name="s", num_cores=NC, num_subcores=NS
    )
    out_type = (
        jax.ShapeDtypeStruct((B, H, 144), _f32),
        jax.ShapeDtypeStruct((B, H, 144), _f32),
        jax.ShapeDtypeStruct((B, 144), _f32),
        jax.ShapeDtypeStruct((B, 8), _f32),
        jax.ShapeDtypeStruct((B, 32), _f32),
        jax.ShapeDtypeStruct((B, 16), _f32),
    )
    idx_set = [
        pltpu.VMEM((CY,), _i32),
        pltpu.VMEM((CY,), _i32),
        pltpu.VMEM((CY, 5), _i32),
        pltpu.VMEM((CY, 5), _i32),
        pltpu.VMEM((CY, 10), _i32),
        pltpu.VMEM((CY, 5), _i32),
        pltpu.VMEM((5 * CY,), _i32),
        pltpu.VMEM((5 * CY,), _i32),
        pltpu.VMEM((10 * CY,), _i32),
        pltpu.VMEM((5 * CY,), _i32),
    ]
    g_set = [
        pltpu.VMEM((CY, 32), _f32),
        pltpu.VMEM((CY, 16), _f32),
        pltpu.VMEM((5 * CY, 16), _f32),
        pltpu.VMEM((5 * CY, 16), _f32),
        pltpu.VMEM((10 * CY, 16), _f32),
        pltpu.VMEM((5 * CY, 16), _f32),
        pltpu.VMEM((CY, 32), _f32),
    ]
    scratch = (
        idx_set + idx_set + g_set + g_set
        + [pltpu.VMEM((CY, 144), _f32)] * 2
        + [pltpu.VMEM((CY, 8), _f32),
           pltpu.VMEM((CY, 32), _f32),
           pltpu.VMEM((CY, 16), _f32)]
        + [pltpu.SemaphoreType.DMA] * 6
    )

    @functools.partial(
        pl.kernel, mesh=mesh, out_type=out_type, scratch_types=scratch,
        name="ps_embed_sc",
        compiler_params=pltpu.CompilerParams(use_tc_tiling_on_sc=False, needs_layout_passes=False),
    )
    def k(
        l_xs, l_dom, l_cate, l_chn, l_seg, l_pid, l_sem,
        n_xs, n_dom, n_cate, n_chn, n_seg, n_pid, n_sem,
        ys, y_dom, y_cate, y_chn, y_seg, y_pid, y_sem,
        os_i, uid_i, zip_i,
        W_domain, W_cate, W_chn, W_seg, W_os, W_doc, W_uid, W_zip, W_pid,
        o_lx, o_nx, o_y, o_os, o_uid, o_zip,
        ia0, ib0, ic20, ih20, is20, ip20, ic0, ih0, is0, ip0,
        ia1, ib1, ic21, ih21, is21, ip21, ic1, ih1, is1, ip1,
        gd0, gm0, gc0, gh0, gs0, gp0, ge0,
        gd1, gm1, gc1, gh1, gs1, gp1, ge1,
        ob0, ob1,
        s_os, s_uid, s_zip,
        isem0, isem1, gsem0, gsem1, osem0, osem1,
    ):
        wid = lax.axis_index("s") * NC + lax.axis_index("c")
        IDX = ((ia0, ib0, ic20, ih20, is20, ip20, ic0, ih0, is0, ip0),
               (ia1, ib1, ic21, ih21, is21, ip21, ic1, ih1, is1, ip1))
        G = ((gd0, gm0, gc0, gh0, gs0, gp0, ge0),
             (gd1, gm1, gc1, gh1, gs1, gp1, ge1))
        OB = (ob0, ob1)
        ISEM = (isem0, isem1)
        GSEM = (gsem0, gsem1)
        OSEM = (osem0, osem1)

        def flatten_idx(src2, dst1, total, width):
            lane = lax.iota(_i32, 16)
            rows = jnp.zeros((16,), _i32)
            for th in range(width, 16, width):
                rows = rows + jnp.where(lane >= th, _i32(1), _i32(0))
            cols = lane - rows * _i32(width)
            qstep = 16 // width
            rstep = 16 - qstep * width
            for blk in range((total + 15) // 16):
                f0 = blk * 16
                dst1[pl.ds(f0, 16)] = plsc.load_gather(src2, [rows, cols])
                tmp = cols + _i32(rstep)
                wrap = jnp.where(tmp >= _i32(width), _i32(1), _i32(0))
                cols = tmp - wrap * _i32(width)
                rows = rows + _i32(qstep) + wrap

        def flatten_all(p, n):
            _, _, ic2, ih2, is2, ip2, ic, ih, isg, ip = IDX[p]
            flatten_idx(ic2, ic, 5 * n, 5)
            flatten_idx(ih2, ih, 5 * n, 5)
            flatten_idx(is2, isg, 10 * n, 10)
            flatten_idx(ip2, ip, 5 * n, 5)

        def gather_runs(tbl, iref, gref, tot, sem):
            cps = []
            for j in range(0, tot, 128):
                nrun = min(128, tot - j)
                cps.append(pltpu.make_async_copy(
                    tbl.at[iref.at[pl.ds(j, nrun)]],
                    gref.at[pl.ds(j, nrun)], sem))
            return cps

        def x_idx_copies(bb, p, feats):
            xs_r, dom_r, cate_r, chn_r, seg_r, pid_r = feats
            ia, ib, ic2, ih2, is2, ip2 = IDX[p][:6]
            return [
                pltpu.make_async_copy(xs_r.at[bb], ia.at[pl.ds(0, H)],
                                      ISEM[p]),
                pltpu.make_async_copy(dom_r.at[bb], ib.at[pl.ds(0, H)],
                                      ISEM[p]),
                pltpu.make_async_copy(cate_r.at[bb], ic2.at[pl.ds(0, H)],
                                      ISEM[p]),
                pltpu.make_async_copy(chn_r.at[bb], ih2.at[pl.ds(0, H)],
                                      ISEM[p]),
                pltpu.make_async_copy(seg_r.at[bb], is2.at[pl.ds(0, H)],
                                      ISEM[p]),
                pltpu.make_async_copy(pid_r.at[bb], ip2.at[pl.ds(0, H)],
                                      ISEM[p]),
            ]

        def x_gather_copies(bb, p, sem_r):
            ia, ib = IDX[p][0], IDX[p][1]
            ic, ih, isg, ip = IDX[p][6:]
            gd, gm, gc, gh, gs, gp, ge = G[p]
            sem = GSEM[p]
            cps = [
                pltpu.make_async_copy(W_doc.at[ia.at[pl.ds(0, H)]],
                                      gd.at[pl.ds(0, H)], sem),
                pltpu.make_async_copy(W_domain.at[ib.at[pl.ds(0, H)]],
                                      gm.at[pl.ds(0, H)], sem),
            ]
            cps += gather_runs(W_cate, ic, gc, 5 * H, sem)
            cps += gather_runs(W_chn, ih, gh, 5 * H, sem)
            cps += gather_runs(W_pid, ip, gp, 5 * H, sem)
            cps += gather_runs(W_seg, isg, gs, 10 * H, sem)
            cps.append(pltpu.make_async_copy(sem_r.at[bb],
                                             ge.at[pl.ds(0, H)], sem))
            return cps

        def compute(n, p, q, pid_scale):
            gd, gm, gc, gh, gs, gp, ge = G[p]
            ob = OB[q]

            @pl.loop(0, n)
            def _(i):
                i5 = 5 * i
                i10 = 10 * i
                ob[i, pl.ds(0, 16)] = gd[i, pl.ds(0, 16)]
                ob[i, pl.ds(16, 16)] = gd[i, pl.ds(16, 16)]
                ob[i, pl.ds(32, 16)] = gm[i]
                c = ((gc[i5] + gc[i5 + 1]) + (gc[i5 + 2] + gc[i5 + 3])
                     + gc[i5 + 4])
                ob[i, pl.ds(48, 16)] = c
                h = ((gh[i5] + gh[i5 + 1]) + (gh[i5 + 2] + gh[i5 + 3])
                     + gh[i5 + 4])
                ob[i, pl.ds(64, 16)] = h
                s = (((gs[i10] + gs[i10 + 1]) + (gs[i10 + 2] + gs[i10 + 3]))
                     + ((gs[i10 + 4] + gs[i10 + 5])
                        + (gs[i10 + 6] + gs[i10 + 7]))
                     + (gs[i10 + 8] + gs[i10 + 9]))
                ob[i, pl.ds(80, 16)] = s
                ob[i, pl.ds(96, 16)] = ge[i, pl.ds(0, 16)]
                ob[i, pl.ds(112, 16)] = ge[i, pl.ds(16, 16)]
                pp = ((gp[i5] + gp[i5 + 1]) + (gp[i5 + 2] + gp[i5 + 3])
                      + gp[i5 + 4])
                ob[i, pl.ds(128, 16)] = pp * pid_scale

        def x_out_copy(bb, q, out_r):
            return pltpu.make_async_copy(OB[q].at[pl.ds(0, H)], out_r.at[bb],
                                         OSEM[q])

        for (feats, sem_r, out_r) in (
            ((l_xs, l_dom, l_cate, l_chn, l_seg, l_pid), l_sem, o_lx),
            ((n_xs, n_dom, n_cate, n_chn, n_seg, n_pid), n_sem, o_nx),
        ):
            b0 = wid * BW

            def pre(t, pj):
                for d in x_idx_copies(b0 + t, pj, feats):
                    d.start()
                for d in x_idx_copies(b0 + t, pj, feats):
                    d.wait()
                flatten_all(pj, H)
                for d in x_gather_copies(b0 + t, pj, sem_r):
                    d.start()

            def run(t, pj, with_wait_out, fire_next):
                for d in x_gather_copies(b0 + t, pj, sem_r):
                    d.wait()
                if fire_next:
                    for d in x_idx_copies(b0 + t + 2, pj, feats):
                        d.start()
                if with_wait_out:
                    x_out_copy(b0 + t - 2, pj, out_r).wait()
                compute(H, pj, pj, _f32(0.2))
                x_out_copy(b0 + t, pj, out_r).start()
                if fire_next:
                    for d in x_idx_copies(b0 + t + 2, pj, feats):
                        d.wait()
                    flatten_all(pj, H)
                    for d in x_gather_copies(b0 + t + 2, pj, sem_r):
                        d.start()

            pre(0, 0)
            pre(1, 1)
            run(0, 0, False, True)
            run(1, 1, False, True)

            @pl.loop(2, BW - 2, step=2)
            def _(t):
                run(t, 0, True, True)
                run(t + 1, 1, True, True)

            run(BW - 2, 0, True, False)
            run(BW - 1, 1, True, False)
            x_out_copy(b0 + BW - 2, 0, out_r).wait()
            x_out_copy(b0 + BW - 1, 1, out_r).wait()

        for t in range(NYC):
            base = wid * (CY * NYC) + t * CY
            p = t & 1
            ia, ib, ic2, ih2, is2, ip2, ic, ih, isg, ip = IDX[p]
            gd, gm, gc, gh, gs, gp, ge = G[p]
            sem = GSEM[p]
            cps = [
                pltpu.make_async_copy(ys.at[pl.ds(base, CY)], ia, ISEM[p]),
                pltpu.make_async_copy(y_dom.at[pl.ds(base, CY)], ib, ISEM[p]),
                pltpu.make_async_copy(y_cate.at[pl.ds(base, CY)], ic2,
                                      ISEM[p]),
                pltpu.make_async_copy(y_chn.at[pl.ds(base, CY)], ih2,
                                      ISEM[p]),
                pltpu.make_async_copy(y_seg.at[pl.ds(base, CY)], is2,
                                      ISEM[p]),
                pltpu.make_async_copy(y_pid.at[pl.ds(base, CY)], ip2,
                                      ISEM[p]),
            ]
            for d in cps:
                d.start()
            for d in cps:
                d.wait()
            flatten_all(p, CY)
            gcps = [
                pltpu.make_async_copy(W_doc.at[ia], gd, sem),
                pltpu.make_async_copy(W_domain.at[ib], gm, sem),
            ]
            gcps += gather_runs(W_cate, ic, gc, 5 * CY, sem)
            gcps += gather_runs(W_chn, ih, gh, 5 * CY, sem)
            gcps += gather_runs(W_pid, ip, gp, 5 * CY, sem)
            gcps += gather_runs(W_seg, isg, gs, 10 * CY, sem)
            gcps.append(pltpu.make_async_copy(y_sem.at[pl.ds(base, CY)], ge,
                                              sem))
            for d in gcps:
                d.start()
            for d in gcps:
                d.wait()
            compute(CY, p, p, _f32(1.0))
            pltpu.sync_copy(OB[p], o_y.at[pl.ds(base, CY)])

        for t in range(NYC):
            base = wid * (CY * NYC) + t * CY
            ia, ib, ic = IDX[0][0], IDX[0][1], IDX[1][0]
            pltpu.sync_copy(os_i.at[pl.ds(base, CY)], ia)
            pltpu.sync_copy(uid_i.at[pl.ds(base, CY)], ib)
            pltpu.sync_copy(zip_i.at[pl.ds(base, CY)], ic)

            @pl.loop(0, CY // 16)
            def _(kk):
                v = ic[pl.ds(kk * 16, 16)]
                ic[pl.ds(kk * 16, 16)] = jnp.minimum(
                    jnp.maximum(v, 0), jnp.int32(99999))

            d1 = pltpu.async_copy(W_os.at[ia], s_os, gsem0)
            d2 = pltpu.async_copy(W_uid.at[ib], s_uid, gsem0)
            d3 = pltpu.async_copy(W_zip.at[ic], s_zip, gsem0)
            d1.wait(); d2.wait(); d3.wait()
            pltpu.sync_copy(s_os, o_os.at[pl.ds(base, CY)])
            pltpu.sync_copy(s_uid, o_uid.at[pl.ds(base, CY)])
            pltpu.sync_copy(s_zip, o_zip.at[pl.ds(base, CY)])

    return k


_KERNEL = None


def _get_kernel():
    global _KERNEL
    if _KERNEL is None:
        _KERNEL = _make_kernel()
    return _KERNEL


def kernel(local_xs, local_x_domain, local_x_cate, local_x_chn,
           local_x_seg_title, local_x_pid, local_x_semantic_emb,
           nonlocal_xs, nonlocal_x_domain, nonlocal_x_cate, nonlocal_x_chn,
           nonlocal_x_seg_title, nonlocal_x_pid, nonlocal_x_semantic_emb,
           ys, y_domain, y_cate, y_chn, y_seg_title, y_pid, y_semantic_emb,
           os, hashed_uid, zip_code,
           W_domain, W_cate, W_chn, W_seg_title, W_os, W_doc, W_uid, W_zip,
           W_pid):
    k = _get_kernel()
    return k(
        local_xs, local_x_domain, local_x_cate, local_x_chn,
        local_x_seg_title, local_x_pid, local_x_semantic_emb,
        nonlocal_xs, nonlocal_x_domain, nonlocal_x_cate, nonlocal_x_chn,
        nonlocal_x_seg_title, nonlocal_x_pid, nonlocal_x_semantic_emb,
        ys, y_domain, y_cate, y_chn, y_seg_title, y_pid, y_semantic_emb,
        os, hashed_uid, zip_code,
        W_domain, W_cate, W_chn, W_seg_title, W_os, W_doc, W_uid, W_zip,
        W_pid,
    )


# speedup vs baseline: 1.0433x; 1.0433x over previous
import functools

import jax
import jax.numpy as jnp
from jax import lax
from jax.experimental import pallas as pl
from jax.experimental.pallas import tpu as pltpu
from jax.experimental.pallas import tpu_sc as plsc

B = 4096
H = 50
N = B * H
NC, NS = 2, 16
NW = NC * NS
CX = 64
RPW = N // NW
NCHUNK = RPW // CX
BPW = B // NW

_f32 = jnp.float32
_i32 = jnp.int32


def _make_kernel():
    mesh = plsc.VectorSubcoreMesh(
        core_axis_name="c", subcore_axis_name="s", num_cores=NC, num_subcores=NS
    )
    out_type = (
        jax.ShapeDtypeStruct((N * 96,), _f32),
        jax.ShapeDtypeStruct((N * 16,), _f32),
        jax.ShapeDtypeStruct((N * 96,), _f32),
        jax.ShapeDtypeStruct((N * 16,), _f32),
        jax.ShapeDtypeStruct((B * 144,), _f32),
        jax.ShapeDtypeStruct((B, 8), _f32),
        jax.ShapeDtypeStruct((B, 32), _f32),
        jax.ShapeDtypeStruct((B, 16), _f32),
    )
    idx_set = [
        pltpu.VMEM((CX,), _i32),
        pltpu.VMEM((CX,), _i32),
        pltpu.VMEM((5 * CX,), _i32),
        pltpu.VMEM((5 * CX,), _i32),
        pltpu.VMEM((10 * CX,), _i32),
        pltpu.VMEM((5 * CX,), _i32),
    ]
    g_set = [
        pltpu.VMEM((CX, 32), _f32),
        pltpu.VMEM((CX, 16), _f32),
        pltpu.VMEM((5 * CX, 16), _f32),
        pltpu.VMEM((5 * CX, 16), _f32),
        pltpu.VMEM((10 * CX, 16), _f32),
        pltpu.VMEM((5 * CX, 16), _f32),
        pltpu.VMEM((CX * 32,), _f32),
    ]
    scratch = (
        idx_set + idx_set + g_set + g_set
        + [pltpu.VMEM((CX * 144,), _f32)] * 2
        + [pltpu.VMEM((CX * 96,), _f32)] * 2
        + [pltpu.VMEM((CX * 16,), _f32)] * 2
        + [pltpu.VMEM((CX, 8), _f32),
           pltpu.VMEM((CX, 32), _f32),
           pltpu.VMEM((CX, 16), _f32)]
        + [pltpu.SemaphoreType.DMA] * 6
    )

    @functools.partial(
        pl.kernel, mesh=mesh, out_type=out_type, scratch_types=scratch,
        name="ps_embed_sc",
        compiler_params=pltpu.CompilerParams(use_tc_tiling_on_sc=False),
    )
    def k(
        l_xs, l_dom, l_cate, l_chn, l_seg, l_pid,
        n_xs, n_dom, n_cate, n_chn, n_seg, n_pid,
        ys, y_dom, y_cate, y_chn, y_seg, y_pid, y_sem,
        os_i, uid_i, zip_i,
        W_domain, W_cate, W_chn, W_seg, W_os, W_doc, W_uid, W_zip, W_pid,
        o_lx1, o_lx2, o_nx1, o_nx2, o_y, o_os, o_uid, o_zip,
        ia0, ib0, ic0, ih0, is0, ip0,
        ia1, ib1, ic1, ih1, is1, ip1,
        gd0, gm0, gc0, gh0, gs0, gp0, ge0,
        gd1, gm1, gc1, gh1, gs1, gp1, ge1,
        ob0, ob1, xb0, xb1, pb0, pb1,
        s_os, s_uid, s_zip,
        isem0, isem1, gsem0, gsem1, osem0, osem1,
    ):
        wid = lax.axis_index("s") * NC + lax.axis_index("c")
        IDX = ((ia0, ib0, ic0, ih0, is0, ip0), (ia1, ib1, ic1, ih1, is1, ip1))
        G = ((gd0, gm0, gc0, gh0, gs0, gp0, ge0),
             (gd1, gm1, gc1, gh1, gs1, gp1, ge1))
        OB = (ob0, ob1)
        XB = (xb0, xb1)
        PB = (pb0, pb1)
        ISEM = (isem0, isem1)
        GSEM = (gsem0, gsem1)
        OSEM = (osem0, osem1)

        def idx_copies(base, p, feats):
            xs_r, dom_r, cate_r, chn_r, seg_r, pid_r = feats
            ia, ib, ic, ih, isg, ip = IDX[p]
            return [
                pltpu.make_async_copy(xs_r.at[pl.ds(base, CX)], ia, ISEM[p]),
                pltpu.make_async_copy(dom_r.at[pl.ds(base, CX)], ib, ISEM[p]),
                pltpu.make_async_copy(cate_r.at[pl.ds(5 * base, 5 * CX)], ic,
                                      ISEM[p]),
                pltpu.make_async_copy(chn_r.at[pl.ds(5 * base, 5 * CX)], ih,
                                      ISEM[p]),
                pltpu.make_async_copy(seg_r.at[pl.ds(10 * base, 10 * CX)], isg,
                                      ISEM[p]),
                pltpu.make_async_copy(pid_r.at[pl.ds(5 * base, 5 * CX)], ip,
                                      ISEM[p]),
            ]

        def gather_copies(base, p, sem_r, tables):
            W_dom_r, W_cate_r, W_chn_r, W_seg_r, W_doc_r, W_pid_r = tables
            ia, ib, ic, ih, isg, ip = IDX[p]
            gd, gm, gc, gh, gs, gp, ge = G[p]
            sem = GSEM[p]
            cps = [
                pltpu.make_async_copy(W_doc_r.at[ia], gd, sem),
                pltpu.make_async_copy(W_dom_r.at[ib], gm, sem),
            ]
            for j in range(0, 5 * CX, 128):
                nrun = min(128, 5 * CX - j)
                cps.append(pltpu.make_async_copy(
                    W_cate_r.at[ic.at[pl.ds(j, nrun)]],
                    gc.at[pl.ds(j, nrun)], sem))
                cps.append(pltpu.make_async_copy(
                    W_chn_r.at[ih.at[pl.ds(j, nrun)]],
                    gh.at[pl.ds(j, nrun)], sem))
                cps.append(pltpu.make_async_copy(
                    W_pid_r.at[ip.at[pl.ds(j, nrun)]],
                    gp.at[pl.ds(j, nrun)], sem))
            for j in range(0, 10 * CX, 128):
                nrun = min(128, 10 * CX - j)
                cps.append(pltpu.make_async_copy(
                    W_seg_r.at[isg.at[pl.ds(j, nrun)]],
                    gs.at[pl.ds(j, nrun)], sem))
            if sem_r is not None:
                cps.append(pltpu.make_async_copy(
                    sem_r.at[pl.ds(32 * base, 32 * CX)], ge, sem))
            return cps

        def compute_x(p, q):
            gd, gm, gc, gh, gs, gp, ge = G[p]
            xb = XB[q]
            pb = PB[q]

            @pl.loop(0, CX)
            def _(i):
                off = 96 * i
                i5 = 5 * i
                i10 = 10 * i
                xb[pl.ds(off, 16)] = gd[i, pl.ds(0, 16)]
                xb[pl.ds(off + 16, 16)] = gd[i, pl.ds(16, 16)]
                xb[pl.ds(off + 32, 16)] = gm[i]
                c = ((gc[i5] + gc[i5 + 1]) + (gc[i5 + 2] + gc[i5 + 3])
                     + gc[i5 + 4])
                xb[pl.ds(off + 48, 16)] = c
                h = ((gh[i5] + gh[i5 + 1]) + (gh[i5 + 2] + gh[i5 + 3])
                     + gh[i5 + 4])
                xb[pl.ds(off + 64, 16)] = h
                s = (((gs[i10] + gs[i10 + 1]) + (gs[i10 + 2] + gs[i10 + 3]))
                     + ((gs[i10 + 4] + gs[i10 + 5])
                        + (gs[i10 + 6] + gs[i10 + 7]))
                     + (gs[i10 + 8] + gs[i10 + 9]))
                xb[pl.ds(off + 80, 16)] = s
                pp = ((gp[i5] + gp[i5 + 1]) + (gp[i5 + 2] + gp[i5 + 3])
                      + gp[i5 + 4])
                pb[pl.ds(16 * i, 16)] = pp * _f32(0.2)

        def compute(p, q, pid_scale):
            gd, gm, gc, gh, gs, gp, ge = G[p]
            ob = OB[q]

            @pl.loop(0, CX)
            def _(i):
                off = 144 * i
                i5 = 5 * i
                i10 = 10 * i
                ob[pl.ds(off, 16)] = gd[i, pl.ds(0, 16)]
                ob[pl.ds(off + 16, 16)] = gd[i, pl.ds(16, 16)]
                ob[pl.ds(off + 32, 16)] = gm[i]
                c = ((gc[i5] + gc[i5 + 1]) + (gc[i5 + 2] + gc[i5 + 3])
                     + gc[i5 + 4])
                ob[pl.ds(off + 48, 16)] = c
                h = ((gh[i5] + gh[i5 + 1]) + (gh[i5 + 2] + gh[i5 + 3])
                     + gh[i5 + 4])
                ob[pl.ds(off + 64, 16)] = h
                s = (((gs[i10] + gs[i10 + 1]) + (gs[i10 + 2] + gs[i10 + 3]))
                     + ((gs[i10 + 4] + gs[i10 + 5])
                        + (gs[i10 + 6] + gs[i10 + 7]))
                     + (gs[i10 + 8] + gs[i10 + 9]))
                ob[pl.ds(off + 80, 16)] = s
                ob[pl.ds(off + 96, 16)] = ge[pl.ds(32 * i, 16)]
                ob[pl.ds(off + 112, 16)] = ge[pl.ds(32 * i + 16, 16)]
                pp = ((gp[i5] + gp[i5 + 1]) + (gp[i5 + 2] + gp[i5 + 3])
                      + gp[i5 + 4])
                ob[pl.ds(off + 128, 16)] = pp * pid_scale

        def out_copy(base, q, out_r):
            return pltpu.make_async_copy(
                OB[q], out_r.at[pl.ds(144 * base, 144 * CX)], OSEM[q])

        def x_out_copies(base, q, out1_r, out2_r):
            return [
                pltpu.make_async_copy(
                    XB[q], out1_r.at[pl.ds(96 * base, 96 * CX)], OSEM[q]),
                pltpu.make_async_copy(
                    PB[q], out2_r.at[pl.ds(16 * base, 16 * CX)], OSEM[q]),
            ]

        TABLES = (W_domain, W_cate, W_chn, W_seg, W_doc, W_pid)

        for (feats, out1_r, out2_r) in (
            ((l_xs, l_dom, l_cate, l_chn, l_seg, l_pid), o_lx1, o_lx2),
            ((n_xs, n_dom, n_cate, n_chn, n_seg, n_pid), o_nx1, o_nx2),
        ):
            x0 = wid * RPW

            def pre(t, pj):
                for d in idx_copies(x0 + t * CX, pj, feats):
                    d.start()
                for d in idx_copies(x0 + t * CX, pj, feats):
                    d.wait()
                for d in gather_copies(x0 + t * CX, pj, None, TABLES):
                    d.start()

            def run(t, pj, with_wait_out, fire_next):
                for d in gather_copies(x0 + t * CX, pj, None, TABLES):
                    d.wait()
                if fire_next:
                    for d in idx_copies(x0 + (t + 2) * CX, pj, feats):
                        d.start()
                if with_wait_out:
                    for d in x_out_copies(x0 + (t - 2) * CX, pj, out1_r,
                                          out2_r):
                        d.wait()
                compute_x(pj, pj)
                for d in x_out_copies(x0 + t * CX, pj, out1_r, out2_r):
                    d.start()
                if fire_next:
                    for d in idx_copies(x0 + (t + 2) * CX, pj, feats):
                        d.wait()
                    for d in gather_copies(x0 + (t + 2) * CX, pj, None,
                                           TABLES):
                        d.start()

            pre(0, 0)
            pre(1, 1)
            run(0, 0, False, True)
            run(1, 1, False, True)

            @pl.loop(2, NCHUNK - 2, step=2)
            def _(t):
                run(t, 0, True, True)
                run(t + 1, 1, True, True)

            run(NCHUNK - 2, 0, True, False)
            run(NCHUNK - 1, 1, True, False)
            for d in x_out_copies(x0 + (NCHUNK - 2) * CX, 0, out1_r, out2_r):
                d.wait()
            for d in x_out_copies(x0 + (NCHUNK - 1) * CX, 1, out1_r, out2_r):
                d.wait()

        yfeats = (ys, y_dom, y_cate, y_chn, y_seg, y_pid)
        for t in range(2):
            base = wid * BPW + t * CX
            p = t & 1
            for d in idx_copies(base, p, yfeats):
                d.start()
            for d in idx_copies(base, p, yfeats):
                d.wait()
            for d in gather_copies(base, p, y_sem, TABLES):
                d.start()
            for d in gather_copies(base, p, y_sem, TABLES):
                d.wait()
            compute(p, p, _f32(1.0))
            out_copy(base, p, o_y).start()
            out_copy(base, p, o_y).wait()

        for t in range(2):
            base = wid * BPW + t * CX
            ia, ib, ic = IDX[0][0], IDX[0][1], IDX[1][0]
            pltpu.sync_copy(os_i.at[pl.ds(base, CX)], ia)
            pltpu.sync_copy(uid_i.at[pl.ds(base, CX)], ib)
            pltpu.sync_copy(zip_i.at[pl.ds(base, CX)], ic)

            @pl.loop(0, CX // 16)
            def _(kk):
                v = ic[pl.ds(kk * 16, 16)]
                ic[pl.ds(kk * 16, 16)] = jnp.minimum(
                    jnp.maximum(v, 0), jnp.int32(99999))

            d1 = pltpu.async_copy(W_os.at[ia], s_os, gsem0)
            d2 = pltpu.async_copy(W_uid.at[ib], s_uid, gsem0)
            d3 = pltpu.async_copy(W_zip.at[ic], s_zip, gsem0)
            d1.wait(); d2.wait(); d3.wait()
            pltpu.sync_copy(s_os, o_os.at[pl.ds(base, CX)])
            pltpu.sync_copy(s_uid, o_uid.at[pl.ds(base, CX)])
            pltpu.sync_copy(s_zip, o_zip.at[pl.ds(base, CX)])

    return k


_KERNEL = None


def _get_kernel():
    global _KERNEL
    if _KERNEL is None:
        _KERNEL = _make_kernel()
    return _KERNEL


def kernel(local_xs, local_x_domain, local_x_cate, local_x_chn,
           local_x_seg_title, local_x_pid, local_x_semantic_emb,
           nonlocal_xs, nonlocal_x_domain, nonlocal_x_cate, nonlocal_x_chn,
           nonlocal_x_seg_title, nonlocal_x_pid, nonlocal_x_semantic_emb,
           ys, y_domain, y_cate, y_chn, y_seg_title, y_pid, y_semantic_emb,
           os, hashed_uid, zip_code,
           W_domain, W_cate, W_chn, W_seg_title, W_os, W_doc, W_uid, W_zip,
           W_pid):
    k = _get_kernel()
    flat = lambda a: a.reshape(-1)
    lx1, lx2, nx1, nx2, y, os_e, uid_e, zip_e = k(
        flat(local_xs), flat(local_x_domain), flat(local_x_cate),
        flat(local_x_chn), flat(local_x_seg_title), flat(local_x_pid),
        flat(nonlocal_xs), flat(nonlocal_x_domain), flat(nonlocal_x_cate),
        flat(nonlocal_x_chn), flat(nonlocal_x_seg_title), flat(nonlocal_x_pid),
        flat(ys), flat(y_domain), flat(y_cate), flat(y_chn),
        flat(y_seg_title), flat(y_pid), flat(y_semantic_emb),
        flat(os), flat(hashed_uid), flat(zip_code),
        W_domain, W_cate, W_chn, W_seg_title, W_os, W_doc, W_uid, W_zip,
        W_pid,
    )
    lx = jnp.concatenate(
        [lx1.reshape(B, H, 96), local_x_semantic_emb.astype(_f32),
         lx2.reshape(B, H, 16)], axis=-1)
    nx = jnp.concatenate(
        [nx1.reshape(B, H, 96), nonlocal_x_semantic_emb.astype(_f32),
         nx2.reshape(B, H, 16)], axis=-1)
    return (lx, nx, y.reshape(B, 144), os_e, uid_e, zip_e)


# speedup vs baseline: 1.0743x; 1.0297x over previous
import functools

import jax
import jax.numpy as jnp
from jax import lax
from jax.experimental import pallas as pl
from jax.experimental.pallas import tpu as pltpu
from jax.experimental.pallas import tpu_sc as plsc

B = 4096
H = 50
N = B * H
NC, NS = 2, 16
NW = NC * NS
CX = 64
RPW = N // NW
NCHUNK = RPW // CX
BPW = B // NW

_f32 = jnp.float32
_i32 = jnp.int32


def _mesh():
    return plsc.VectorSubcoreMesh(
        core_axis_name="c", subcore_axis_name="s", num_cores=NC, num_subcores=NS
    )


def _make_kernel1():
    out_type = (
        jax.ShapeDtypeStruct((N, 32), _f32),
        jax.ShapeDtypeStruct((N, 16), _f32),
        jax.ShapeDtypeStruct((N, 32), _f32),
        jax.ShapeDtypeStruct((N, 16), _f32),
        jax.ShapeDtypeStruct((B * 144,), _f32),
        jax.ShapeDtypeStruct((B, 8), _f32),
        jax.ShapeDtypeStruct((B, 32), _f32),
        jax.ShapeDtypeStruct((B, 16), _f32),
    )
    scratch = (
        [pltpu.VMEM((CX,), _i32)] * 6
        + [pltpu.VMEM((CX, 32), _f32),
           pltpu.VMEM((CX, 16), _f32),
           pltpu.VMEM((CX, 32), _f32),
           pltpu.VMEM((CX, 16), _f32),
           pltpu.VMEM((CX, 32), _f32),
           pltpu.VMEM((CX, 16), _f32)]
        + [pltpu.VMEM((5 * CX,), _i32)] * 3
        + [pltpu.VMEM((10 * CX,), _i32)]
        + [pltpu.VMEM((5 * CX, 16), _f32),
           pltpu.VMEM((5 * CX, 16), _f32),
           pltpu.VMEM((10 * CX, 16), _f32),
           pltpu.VMEM((5 * CX, 16), _f32),
           pltpu.VMEM((CX * 32,), _f32)]
        + [pltpu.VMEM((CX * 144,), _f32)]
        + [pltpu.VMEM((CX, 8), _f32),
           pltpu.VMEM((CX, 32), _f32),
           pltpu.VMEM((CX, 16), _f32)]
        + [pltpu.SemaphoreType.DMA] * 9
    )

    @functools.partial(
        pl.kernel, mesh=_mesh(), out_type=out_type, scratch_types=scratch,
        name="ps_embed_sc_p1",
        compiler_params=pltpu.CompilerParams(use_tc_tiling_on_sc=False),
    )
    def k(
        l_xs, l_dom, n_xs, n_dom,
        ys, y_dom, y_cate, y_chn, y_seg, y_pid, y_sem,
        os_i, uid_i, zip_i,
        W_domain, W_cate, W_chn, W_seg, W_os, W_doc, W_uid, W_zip, W_pid,
        o_ldoc, o_ldom, o_ndoc, o_ndom, o_y, o_os, o_uid, o_zip,
        ia0, ib0, ia1, ib1, ia2, ib2,
        gd0, gm0, gd1, gm1, gd2, gm2,
        icy, ihy, ipy, isy,
        gcy, ghy, gsy, gpy, gey,
        oby,
        s_os, s_uid, s_zip,
        isem0, isem1, isem2, gsem0, gsem1, gsem2, osem0, osem1, osem2,
    ):
        wid = lax.axis_index("s") * NC + lax.axis_index("c")
        IA = (ia0, ia1, ia2)
        IB = (ib0, ib1, ib2)
        GD = (gd0, gd1, gd2)
        GM = (gm0, gm1, gm2)
        ISEM = (isem0, isem1, isem2)
        GSEM = (gsem0, gsem1, gsem2)
        OSEM = (osem0, osem1, osem2)

        def idx_copies(base, p, xs_r, dom_r):
            return [
                pltpu.make_async_copy(xs_r.at[pl.ds(base, CX)], IA[p],
                                      ISEM[p]),
                pltpu.make_async_copy(dom_r.at[pl.ds(base, CX)], IB[p],
                                      ISEM[p]),
            ]

        def gather_copies(p):
            return [
                pltpu.make_async_copy(W_doc.at[IA[p]], GD[p], GSEM[p]),
                pltpu.make_async_copy(W_domain.at[IB[p]], GM[p], GSEM[p]),
            ]

        def out_copies(base, p, od_r, om_r):
            return [
                pltpu.make_async_copy(
                    GD[p], od_r.at[pl.ds(base, CX)], OSEM[p]),
                pltpu.make_async_copy(
                    GM[p], om_r.at[pl.ds(base, CX)], OSEM[p]),
            ]

        for (xs_r, dom_r, od_r, om_r) in (
            (l_xs, l_dom, o_ldoc, o_ldom),
            (n_xs, n_dom, o_ndoc, o_ndom),
        ):
            x0 = wid * RPW

            def run(t, p, fire_idx3, fire_g2, skip_out_wait=False):
                q = (p + 2) % 3
                for d in gather_copies(p):
                    d.wait()
                for d in out_copies(x0 + t * CX, p, od_r, om_r):
                    d.start()
                if fire_idx3:
                    for d in idx_copies(x0 + (t + 3) * CX, p, xs_r, dom_r):
                        d.start()
                if fire_g2:
                    if not skip_out_wait:
                        for d in out_copies(x0 + (t - 1) * CX, q, od_r,
                                            om_r):
                            d.wait()
                    for d in idx_copies(x0 + (t + 2) * CX, q, xs_r, dom_r):
                        d.wait()
                    for d in gather_copies(q):
                        d.start()

            for t0 in range(3):
                for d in idx_copies(x0 + t0 * CX, t0, xs_r, dom_r):
                    d.start()
            for t0 in range(2):
                for d in idx_copies(x0 + t0 * CX, t0, xs_r, dom_r):
                    d.wait()
            for p0 in (0, 1):
                for d in gather_copies(p0):
                    d.start()

            run(0, 0, True, True, skip_out_wait=True)
            run(1, 1, True, True)

            @pl.loop(2, NCHUNK - 5, step=3)
            def _(t):
                run(t, 2, True, True)
                run(t + 1, 0, True, True)
                run(t + 2, 1, True, True)

            run(NCHUNK - 5, 2, True, True)
            run(NCHUNK - 4, 0, True, True)
            run(NCHUNK - 3, 1, False, True)
            run(NCHUNK - 2, 2, False, False)
            run(NCHUNK - 1, 0, False, False)
            for t0 in (NCHUNK - 3, NCHUNK - 2, NCHUNK - 1):
                for d in out_copies(x0 + t0 * CX, t0 % 3, od_r, om_r):
                    d.wait()

        for t in range(2):
            base = wid * BPW + t * CX
            cps = [
                pltpu.make_async_copy(ys.at[pl.ds(base, CX)], ia0, isem0),
                pltpu.make_async_copy(y_dom.at[pl.ds(base, CX)], ib0, isem0),
                pltpu.make_async_copy(y_cate.at[pl.ds(5 * base, 5 * CX)],
                                      icy, isem0),
                pltpu.make_async_copy(y_chn.at[pl.ds(5 * base, 5 * CX)],
                                      ihy, isem0),
                pltpu.make_async_copy(y_seg.at[pl.ds(10 * base, 10 * CX)],
                                      isy, isem0),
                pltpu.make_async_copy(y_pid.at[pl.ds(5 * base, 5 * CX)],
                                      ipy, isem0),
            ]
            for d in cps:
                d.start()
            for d in cps:
                d.wait()
            gcps = [
                pltpu.make_async_copy(W_doc.at[ia0], gd0, gsem0),
                pltpu.make_async_copy(W_domain.at[ib0], gm0, gsem0),
            ]
            for tbl, iref, gref, tot in ((W_cate, icy, gcy, 5 * CX),
                                         (W_chn, ihy, ghy, 5 * CX),
                                         (W_pid, ipy, gpy, 5 * CX),
                                         (W_seg, isy, gsy, 10 * CX)):
                for j in range(0, tot, 128):
                    nrun = min(128, tot - j)
                    gcps.append(pltpu.make_async_copy(
                        tbl.at[iref.at[pl.ds(j, nrun)]],
                        gref.at[pl.ds(j, nrun)], gsem0))
            gcps.append(pltpu.make_async_copy(
                y_sem.at[pl.ds(32 * base, 32 * CX)], gey, gsem0))
            for d in gcps:
                d.start()
            for d in gcps:
                d.wait()

            @pl.loop(0, CX)
            def _(i):
                off = 144 * i
                i5 = 5 * i
                i10 = 10 * i
                oby[pl.ds(off, 16)] = gd0[i, pl.ds(0, 16)]
                oby[pl.ds(off + 16, 16)] = gd0[i, pl.ds(16, 16)]
                oby[pl.ds(off + 32, 16)] = gm0[i]
                c = ((gcy[i5] + gcy[i5 + 1]) + (gcy[i5 + 2] + gcy[i5 + 3])
                     + gcy[i5 + 4])
                oby[pl.ds(off + 48, 16)] = c
                h = ((ghy[i5] + ghy[i5 + 1]) + (ghy[i5 + 2] + ghy[i5 + 3])
                     + ghy[i5 + 4])
                oby[pl.ds(off + 64, 16)] = h
                s = (((gsy[i10] + gsy[i10 + 1]) + (gsy[i10 + 2]
                                                  + gsy[i10 + 3]))
                     + ((gsy[i10 + 4] + gsy[i10 + 5])
                        + (gsy[i10 + 6] + gsy[i10 + 7]))
                     + (gsy[i10 + 8] + gsy[i10 + 9]))
                oby[pl.ds(off + 80, 16)] = s
                oby[pl.ds(off + 96, 16)] = gey[pl.ds(32 * i, 16)]
                oby[pl.ds(off + 112, 16)] = gey[pl.ds(32 * i + 16, 16)]
                pp = ((gpy[i5] + gpy[i5 + 1]) + (gpy[i5 + 2] + gpy[i5 + 3])
                      + gpy[i5 + 4])
                oby[pl.ds(off + 128, 16)] = pp
            pltpu.sync_copy(oby, o_y.at[pl.ds(144 * base, 144 * CX)])

        for t in range(2):
            base = wid * BPW + t * CX
            pltpu.sync_copy(os_i.at[pl.ds(base, CX)], ia0)
            pltpu.sync_copy(uid_i.at[pl.ds(base, CX)], ib0)
            pltpu.sync_copy(zip_i.at[pl.ds(base, CX)], ia1)

            @pl.loop(0, CX // 16)
            def _(kk):
                v = ia1[pl.ds(kk * 16, 16)]
                ia1[pl.ds(kk * 16, 16)] = jnp.minimum(
                    jnp.maximum(v, 0), jnp.int32(99999))

            d1 = pltpu.async_copy(W_os.at[ia0], s_os, gsem0)
            d2 = pltpu.async_copy(W_uid.at[ib0], s_uid, gsem0)
            d3 = pltpu.async_copy(W_zip.at[ia1], s_zip, gsem0)
            d1.wait(); d2.wait(); d3.wait()
            pltpu.sync_copy(s_os, o_os.at[pl.ds(base, CX)])
            pltpu.sync_copy(s_uid, o_uid.at[pl.ds(base, CX)])
            pltpu.sync_copy(s_zip, o_zip.at[pl.ds(base, CX)])

    return k


def _make_kernel2():
    out_type = (
        jax.ShapeDtypeStruct((N * 144,), _f32),
        jax.ShapeDtypeStruct((N * 144,), _f32),
    )
    idx_set = [
        pltpu.VMEM((5 * CX,), _i32),
        pltpu.VMEM((5 * CX,), _i32),
        pltpu.VMEM((10 * CX,), _i32),
        pltpu.VMEM((5 * CX,), _i32),
    ]
    g_set = [
        pltpu.VMEM((CX, 32), _f32),
        pltpu.VMEM((CX, 16), _f32),
        pltpu.VMEM((5 * CX, 16), _f32),
        pltpu.VMEM((5 * CX, 16), _f32),
        pltpu.VMEM((10 * CX, 16), _f32),
        pltpu.VMEM((5 * CX, 16), _f32),
        pltpu.VMEM((CX * 32,), _f32),
    ]
    scratch = (
        idx_set + idx_set + g_set + g_set
        + [pltpu.VMEM((CX * 144,), _f32)] * 2
        + [pltpu.SemaphoreType.DMA] * 6
    )

    @functools.partial(
        pl.kernel, mesh=_mesh(), out_type=out_type, scratch_types=scratch,
        name="ps_embed_sc_p2",
        compiler_params=pltpu.CompilerParams(use_tc_tiling_on_sc=False),
    )
    def k(
        l_cate, l_chn, l_seg, l_pid, l_sem,
        n_cate, n_chn, n_seg, n_pid, n_sem,
        ldoc, ldom, ndoc, ndom,
        W_cate, W_chn, W_seg, W_pid,
        o_lx, o_nx,
        ic0, ih0, is0, ip0,
        ic1, ih1, is1, ip1,
        gd0, gm0, gc0, gh0, gs0, gp0, ge0,
        gd1, gm1, gc1, gh1, gs1, gp1, ge1,
        ob0, ob1,
        isem0, isem1, gsem0, gsem1, osem0, osem1,
    ):
        wid = lax.axis_index("s") * NC + lax.axis_index("c")
        IDX = ((ic0, ih0, is0, ip0), (ic1, ih1, is1, ip1))
        G = ((gd0, gm0, gc0, gh0, gs0, gp0, ge0),
             (gd1, gm1, gc1, gh1, gs1, gp1, ge1))
        OB = (ob0, ob1)
        ISEM = (isem0, isem1)
        GSEM = (gsem0, gsem1)
        OSEM = (osem0, osem1)

        def idx_copies(base, p, feats):
            cate_r, chn_r, seg_r, pid_r = feats
            ic, ih, isg, ip = IDX[p]
            return [
                pltpu.make_async_copy(cate_r.at[pl.ds(5 * base, 5 * CX)], ic,
                                      ISEM[p]),
                pltpu.make_async_copy(chn_r.at[pl.ds(5 * base, 5 * CX)], ih,
                                      ISEM[p]),
                pltpu.make_async_copy(seg_r.at[pl.ds(10 * base, 10 * CX)],
                                      isg, ISEM[p]),
                pltpu.make_async_copy(pid_r.at[pl.ds(5 * base, 5 * CX)], ip,
                                      ISEM[p]),
            ]

        def gather_copies(base, p, sem_r, doc_r, dom_r):
            ic, ih, isg, ip = IDX[p]
            gd, gm, gc, gh, gs, gp, ge = G[p]
            sem = GSEM[p]
            cps = [
                pltpu.make_async_copy(doc_r.at[pl.ds(base, CX)], gd, sem),
                pltpu.make_async_copy(dom_r.at[pl.ds(base, CX)], gm, sem),
            ]
            for tbl, iref, gref, tot in ((W_cate, ic, gc, 5 * CX),
                                         (W_chn, ih, gh, 5 * CX),
                                         (W_pid, ip, gp, 5 * CX),
                                         (W_seg, isg, gs, 10 * CX)):
                for j in range(0, tot, 128):
                    nrun = min(128, tot - j)
                    cps.append(pltpu.make_async_copy(
                        tbl.at[iref.at[pl.ds(j, nrun)]],
                        gref.at[pl.ds(j, nrun)], sem))
            cps.append(pltpu.make_async_copy(
                sem_r.at[pl.ds(32 * base, 32 * CX)], ge, sem))
            return cps

        def compute(p, q):
            gd, gm, gc, gh, gs, gp, ge = G[p]
            ob = OB[q]

            @pl.loop(0, CX)
            def _(i):
                off = 144 * i
                i5 = 5 * i
                i10 = 10 * i
                ob[pl.ds(off, 16)] = gd[i, pl.ds(0, 16)]
                ob[pl.ds(off + 16, 16)] = gd[i, pl.ds(16, 16)]
                ob[pl.ds(off + 32, 16)] = gm[i]
                c = ((gc[i5] + gc[i5 + 1]) + (gc[i5 + 2] + gc[i5 + 3])
                     + gc[i5 + 4])
                ob[pl.ds(off + 48, 16)] = c
                h = ((gh[i5] + gh[i5 + 1]) + (gh[i5 + 2] + gh[i5 + 3])
                     + gh[i5 + 4])
                ob[pl.ds(off + 64, 16)] = h
                s = (((gs[i10] + gs[i10 + 1]) + (gs[i10 + 2] + gs[i10 + 3]))
                     + ((gs[i10 + 4] + gs[i10 + 5])
                        + (gs[i10 + 6] + gs[i10 + 7]))
                     + (gs[i10 + 8] + gs[i10 + 9]))
                ob[pl.ds(off + 80, 16)] = s
                ob[pl.ds(off + 96, 16)] = ge[pl.ds(32 * i, 16)]
                ob[pl.ds(off + 112, 16)] = ge[pl.ds(32 * i + 16, 16)]
                pp = ((gp[i5] + gp[i5 + 1]) + (gp[i5 + 2] + gp[i5 + 3])
                      + gp[i5 + 4])
                ob[pl.ds(off + 128, 16)] = pp * _f32(0.2)

        def out_copy(base, q, out_r):
            return pltpu.make_async_copy(
                OB[q], out_r.at[pl.ds(144 * base, 144 * CX)], OSEM[q])

        for (feats, sem_r, doc_r, dom_r, out_r) in (
            ((l_cate, l_chn, l_seg, l_pid), l_sem, ldoc, ldom, o_lx),
            ((n_cate, n_chn, n_seg, n_pid), n_sem, ndoc, ndom, o_nx),
        ):
            x0 = wid * RPW

            def pre(t, pj):
                for d in idx_copies(x0 + t * CX, pj, feats):
                    d.start()
                for d in idx_copies(x0 + t * CX, pj, feats):
                    d.wait()
                for d in gather_copies(x0 + t * CX, pj, sem_r, doc_r, dom_r):
                    d.start()

            def run(t, pj, with_wait_out, fire_next):
                for d in gather_copies(x0 + t * CX, pj, sem_r, doc_r, dom_r):
                    d.wait()
                if fire_next:
                    for d in idx_copies(x0 + (t + 2) * CX, pj, feats):
                        d.start()
                if with_wait_out:
                    out_copy(x0 + (t - 2) * CX, pj, out_r).wait()
                compute(pj, pj)
                out_copy(x0 + t * CX, pj, out_r).start()
                if fire_next:
                    for d in idx_copies(x0 + (t + 2) * CX, pj, feats):
                        d.wait()
                    for d in gather_copies(x0 + (t + 2) * CX, pj, sem_r,
                                           doc_r, dom_r):
                        d.start()

            pre(0, 0)
            pre(1, 1)
            run(0, 0, False, True)
            run(1, 1, False, True)

            @pl.loop(2, NCHUNK - 2, step=2)
            def _(t):
                run(t, 0, True, True)
                run(t + 1, 1, True, True)

            run(NCHUNK - 2, 0, True, False)
            run(NCHUNK - 1, 1, True, False)
            out_copy(x0 + (NCHUNK - 2) * CX, 0, out_r).wait()
            out_copy(x0 + (NCHUNK - 1) * CX, 1, out_r).wait()

    return k


_K1 = None
_K2 = None


def _get_kernels():
    global _K1, _K2
    if _K1 is None:
        _K1 = _make_kernel1()
        _K2 = _make_kernel2()
    return _K1, _K2


def kernel(local_xs, local_x_domain, local_x_cate, local_x_chn,
           local_x_seg_title, local_x_pid, local_x_semantic_emb,
           nonlocal_xs, nonlocal_x_domain, nonlocal_x_cate, nonlocal_x_chn,
           nonlocal_x_seg_title, nonlocal_x_pid, nonlocal_x_semantic_emb,
           ys, y_domain, y_cate, y_chn, y_seg_title, y_pid, y_semantic_emb,
           os, hashed_uid, zip_code,
           W_domain, W_cate, W_chn, W_seg_title, W_os, W_doc, W_uid, W_zip,
           W_pid):
    k1, k2 = _get_kernels()
    flat = lambda a: a.reshape(-1)
    ldoc, ldom, ndoc, ndom, y, os_e, uid_e, zip_e = k1(
        flat(local_xs), flat(local_x_domain),
        flat(nonlocal_xs), flat(nonlocal_x_domain),
        flat(ys), flat(y_domain), flat(y_cate), flat(y_chn),
        flat(y_seg_title), flat(y_pid), flat(y_semantic_emb),
        flat(os), flat(hashed_uid), flat(zip_code),
        W_domain, W_cate, W_chn, W_seg_title, W_os, W_doc, W_uid, W_zip,
        W_pid,
    )
    lx, nx = k2(
        flat(local_x_cate), flat(local_x_chn), flat(local_x_seg_title),
        flat(local_x_pid), flat(local_x_semantic_emb),
        flat(nonlocal_x_cate), flat(nonlocal_x_chn),
        flat(nonlocal_x_seg_title), flat(nonlocal_x_pid),
        flat(nonlocal_x_semantic_emb),
        ldoc, ldom, ndoc, ndom,
        W_cate, W_chn, W_seg_title, W_pid,
    )
    return (lx.reshape(B, H, 144), nx.reshape(B, H, 144), y.reshape(B, 144),
            os_e, uid_e, zip_e)


# speedup vs baseline: 1.0862x; 1.0111x over previous
import functools

import jax
import jax.numpy as jnp
from jax import lax
from jax.experimental import pallas as pl
from jax.experimental.pallas import tpu as pltpu
from jax.experimental.pallas import tpu_sc as plsc

B = 4096
H = 50
N = B * H
NC, NS = 2, 16
NW = NC * NS
CX = 64
RPW = N // NW
NCHUNK = RPW // CX
BPW = B // NW

_f32 = jnp.float32
_i32 = jnp.int32


def _make_kernel():
    mesh = plsc.VectorSubcoreMesh(
        core_axis_name="c", subcore_axis_name="s", num_cores=NC, num_subcores=NS
    )
    out_type = (
        jax.ShapeDtypeStruct((N * 144,), _f32),
        jax.ShapeDtypeStruct((N * 144,), _f32),
        jax.ShapeDtypeStruct((B * 144,), _f32),
        jax.ShapeDtypeStruct((B, 8), _f32),
        jax.ShapeDtypeStruct((B, 32), _f32),
        jax.ShapeDtypeStruct((B, 16), _f32),
    )
    idx_set = [
        pltpu.VMEM((CX,), _i32),
        pltpu.VMEM((CX,), _i32),
        pltpu.VMEM((5 * CX,), _i32),
        pltpu.VMEM((5 * CX,), _i32),
        pltpu.VMEM((10 * CX,), _i32),
        pltpu.VMEM((5 * CX,), _i32),
    ]
    g_set = [
        pltpu.VMEM((CX, 32), _f32),
        pltpu.VMEM((CX, 16), _f32),
        pltpu.VMEM((5 * CX, 16), _f32),
        pltpu.VMEM((5 * CX, 16), _f32),
        pltpu.VMEM((10 * CX, 16), _f32),
        pltpu.VMEM((5 * CX, 16), _f32),
        pltpu.VMEM((CX * 32,), _f32),
    ]
    scratch = (
        idx_set + idx_set + g_set + g_set
        + [pltpu.VMEM((CX * 144,), _f32)] * 2
        + [pltpu.VMEM((CX, 8), _f32),
           pltpu.VMEM((CX, 32), _f32),
           pltpu.VMEM((CX, 16), _f32)]
        + [pltpu.SemaphoreType.DMA] * 6
    )

    @functools.partial(
        pl.kernel, mesh=mesh, out_type=out_type, scratch_types=scratch,
        name="ps_embed_sc",
        compiler_params=pltpu.CompilerParams(use_tc_tiling_on_sc=False),
    )
    def k(
        l_xs, l_dom, l_cate, l_chn, l_seg, l_pid, l_sem,
        n_xs, n_dom, n_cate, n_chn, n_seg, n_pid, n_sem,
        ys, y_dom, y_cate, y_chn, y_seg, y_pid, y_sem,
        os_i, uid_i, zip_i,
        W_domain, W_cate, W_chn, W_seg, W_os, W_doc, W_uid, W_zip, W_pid,
        o_lx, o_nx, o_y, o_os, o_uid, o_zip,
        ia0, ib0, ic0, ih0, is0, ip0,
        ia1, ib1, ic1, ih1, is1, ip1,
        gd0, gm0, gc0, gh0, gs0, gp0, ge0,
        gd1, gm1, gc1, gh1, gs1, gp1, ge1,
        ob0, ob1,
        s_os, s_uid, s_zip,
        isem0, isem1, gsem0, gsem1, osem0, osem1,
    ):
        wid = lax.axis_index("s") * NC + lax.axis_index("c")
        IDX = ((ia0, ib0, ic0, ih0, is0, ip0), (ia1, ib1, ic1, ih1, is1, ip1))
        G = ((gd0, gm0, gc0, gh0, gs0, gp0, ge0),
             (gd1, gm1, gc1, gh1, gs1, gp1, ge1))
        OB = (ob0, ob1)
        ISEM = (isem0, isem1)
        GSEM = (gsem0, gsem1)
        OSEM = (osem0, osem1)

        def idx_copies(base, p, feats):
            xs_r, dom_r, cate_r, chn_r, seg_r, pid_r = feats
            ia, ib, ic, ih, isg, ip = IDX[p]
            return [
                pltpu.make_async_copy(xs_r.at[pl.ds(base, CX)], ia, ISEM[p]),
                pltpu.make_async_copy(dom_r.at[pl.ds(base, CX)], ib, ISEM[p]),
                pltpu.make_async_copy(cate_r.at[pl.ds(5 * base, 5 * CX)], ic,
                                      ISEM[p]),
                pltpu.make_async_copy(chn_r.at[pl.ds(5 * base, 5 * CX)], ih,
                                      ISEM[p]),
                pltpu.make_async_copy(seg_r.at[pl.ds(10 * base, 10 * CX)], isg,
                                      ISEM[p]),
                pltpu.make_async_copy(pid_r.at[pl.ds(5 * base, 5 * CX)], ip,
                                      ISEM[p]),
            ]

        def gather_copies(base, p, sem_r, tables):
            W_dom_r, W_cate_r, W_chn_r, W_seg_r, W_doc_r, W_pid_r = tables
            ia, ib, ic, ih, isg, ip = IDX[p]
            gd, gm, gc, gh, gs, gp, ge = G[p]
            sem = GSEM[p]
            cps = [
                pltpu.make_async_copy(W_doc_r.at[ia], gd, sem),
                pltpu.make_async_copy(W_dom_r.at[ib], gm, sem),
            ]
            for j in range(0, 5 * CX, 128):
                nrun = min(128, 5 * CX - j)
                cps.append(pltpu.make_async_copy(
                    W_cate_r.at[ic.at[pl.ds(j, nrun)]],
                    gc.at[pl.ds(j, nrun)], sem))
                cps.append(pltpu.make_async_copy(
                    W_chn_r.at[ih.at[pl.ds(j, nrun)]],
                    gh.at[pl.ds(j, nrun)], sem))
                cps.append(pltpu.make_async_copy(
                    W_pid_r.at[ip.at[pl.ds(j, nrun)]],
                    gp.at[pl.ds(j, nrun)], sem))
            for j in range(0, 10 * CX, 128):
                nrun = min(128, 10 * CX - j)
                cps.append(pltpu.make_async_copy(
                    W_seg_r.at[isg.at[pl.ds(j, nrun)]],
                    gs.at[pl.ds(j, nrun)], sem))
            cps.append(pltpu.make_async_copy(
                sem_r.at[pl.ds(32 * base, 32 * CX)], ge, sem))
            return cps

        def compute(p, q, pid_scale):
            gd, gm, gc, gh, gs, gp, ge = G[p]
            ob = OB[q]

            @pl.loop(0, CX)
            def _(i):
                off = 144 * i
                i5 = 5 * i
                i10 = 10 * i
                ob[pl.ds(off, 16)] = gd[i, pl.ds(0, 16)]
                ob[pl.ds(off + 16, 16)] = gd[i, pl.ds(16, 16)]
                ob[pl.ds(off + 32, 16)] = gm[i]
                c = ((gc[i5] + gc[i5 + 1]) + (gc[i5 + 2] + gc[i5 + 3])
                     + gc[i5 + 4])
                ob[pl.ds(off + 48, 16)] = c
                h = ((gh[i5] + gh[i5 + 1]) + (gh[i5 + 2] + gh[i5 + 3])
                     + gh[i5 + 4])
                ob[pl.ds(off + 64, 16)] = h
                s = (((gs[i10] + gs[i10 + 1]) + (gs[i10 + 2] + gs[i10 + 3]))
                     + ((gs[i10 + 4] + gs[i10 + 5])
                        + (gs[i10 + 6] + gs[i10 + 7]))
                     + (gs[i10 + 8] + gs[i10 + 9]))
                ob[pl.ds(off + 80, 16)] = s
                ob[pl.ds(off + 96, 16)] = ge[pl.ds(32 * i, 16)]
                ob[pl.ds(off + 112, 16)] = ge[pl.ds(32 * i + 16, 16)]
                pp = ((gp[i5] + gp[i5 + 1]) + (gp[i5 + 2] + gp[i5 + 3])
                      + gp[i5 + 4])
                ob[pl.ds(off + 128, 16)] = pp * pid_scale

        def out_copy(base, q, out_r):
            return pltpu.make_async_copy(
                OB[q], out_r.at[pl.ds(144 * base, 144 * CX)], OSEM[q])

        TABLES = (W_domain, W_cate, W_chn, W_seg, W_doc, W_pid)

        for (feats, sem_r, out_r) in (
            ((l_xs, l_dom, l_cate, l_chn, l_seg, l_pid), l_sem, o_lx),
            ((n_xs, n_dom, n_cate, n_chn, n_seg, n_pid), n_sem, o_nx),
        ):
            x0 = wid * RPW

            def pre(t, pj):
                for d in idx_copies(x0 + t * CX, pj, feats):
                    d.start()
                for d in idx_copies(x0 + t * CX, pj, feats):
                    d.wait()
                for d in gather_copies(x0 + t * CX, pj, sem_r, TABLES):
                    d.start()

            def run(t, pj, with_wait_out, fire_next):
                for d in gather_copies(x0 + t * CX, pj, sem_r, TABLES):
                    d.wait()
                if fire_next:
                    for d in idx_copies(x0 + (t + 2) * CX, pj, feats):
                        d.start()
                if with_wait_out:
                    out_copy(x0 + (t - 2) * CX, pj, out_r).wait()
                compute(pj, pj, _f32(0.2))
                out_copy(x0 + t * CX, pj, out_r).start()
                if fire_next:
                    for d in idx_copies(x0 + (t + 2) * CX, pj, feats):
                        d.wait()
                    for d in gather_copies(x0 + (t + 2) * CX, pj, sem_r,
                                           TABLES):
                        d.start()

            pre(0, 0)
            pre(1, 1)
            run(0, 0, False, True)
            run(1, 1, False, True)

            @pl.loop(2, NCHUNK - 2, step=2)
            def _(t):
                run(t, 0, True, True)
                run(t + 1, 1, True, True)

            run(NCHUNK - 2, 0, True, False)
            run(NCHUNK - 1, 1, True, False)
            out_copy(x0 + (NCHUNK - 2) * CX, 0, out_r).wait()
            out_copy(x0 + (NCHUNK - 1) * CX, 1, out_r).wait()

        yfeats = (ys, y_dom, y_cate, y_chn, y_seg, y_pid)
        for t in range(2):
            base = wid * BPW + t * CX
            p = t & 1
            for d in idx_copies(base, p, yfeats):
                d.start()
            for d in idx_copies(base, p, yfeats):
                d.wait()
            for d in gather_copies(base, p, y_sem, TABLES):
                d.start()
            for d in gather_copies(base, p, y_sem, TABLES):
                d.wait()
            compute(p, p, _f32(1.0))
            out_copy(base, p, o_y).start()
            out_copy(base, p, o_y).wait()

        for t in range(2):
            base = wid * BPW + t * CX
            ia, ib, ic = IDX[0][0], IDX[0][1], IDX[1][0]
            pltpu.sync_copy(os_i.at[pl.ds(base, CX)], ia)
            pltpu.sync_copy(uid_i.at[pl.ds(base, CX)], ib)
            pltpu.sync_copy(zip_i.at[pl.ds(base, CX)], ic)

            @pl.loop(0, CX // 16)
            def _(kk):
                v = ic[pl.ds(kk * 16, 16)]
                ic[pl.ds(kk * 16, 16)] = jnp.minimum(
                    jnp.maximum(v, 0), jnp.int32(99999))

            d1 = pltpu.async_copy(W_os.at[ia], s_os, gsem0)
            d2 = pltpu.async_copy(W_uid.at[ib], s_uid, gsem0)
            d3 = pltpu.async_copy(W_zip.at[ic], s_zip, gsem0)
            d1.wait(); d2.wait(); d3.wait()
            pltpu.sync_copy(s_os, o_os.at[pl.ds(base, CX)])
            pltpu.sync_copy(s_uid, o_uid.at[pl.ds(base, CX)])
            pltpu.sync_copy(s_zip, o_zip.at[pl.ds(base, CX)])

    return k


_KERNEL = None


def _get_kernel():
    global _KERNEL
    if _KERNEL is None:
        _KERNEL = _make_kernel()
    return _KERNEL


def kernel(local_xs, local_x_domain, local_x_cate, local_x_chn,
           local_x_seg_title, local_x_pid, local_x_semantic_emb,
           nonlocal_xs, nonlocal_x_domain, nonlocal_x_cate, nonlocal_x_chn,
           nonlocal_x_seg_title, nonlocal_x_pid, nonlocal_x_semantic_emb,
           ys, y_domain, y_cate, y_chn, y_seg_title, y_pid, y_semantic_emb,
           os, hashed_uid, zip_code,
           W_domain, W_cate, W_chn, W_seg_title, W_os, W_doc, W_uid, W_zip,
           W_pid):
    k = _get_kernel()
    flat = lambda a: a.reshape(-1)
    lx, nx, y, os_e, uid_e, zip_e = k(
        flat(local_xs), flat(local_x_domain), flat(local_x_cate),
        flat(local_x_chn), flat(local_x_seg_title), flat(local_x_pid),
        flat(local_x_semantic_emb),
        flat(nonlocal_xs), flat(nonlocal_x_domain), flat(nonlocal_x_cate),
        flat(nonlocal_x_chn), flat(nonlocal_x_seg_title), flat(nonlocal_x_pid),
        flat(nonlocal_x_semantic_emb),
        flat(ys), flat(y_domain), flat(y_cate), flat(y_chn),
        flat(y_seg_title), flat(y_pid), flat(y_semantic_emb),
        flat(os), flat(hashed_uid), flat(zip_code),
        W_domain, W_cate, W_chn, W_seg_title, W_os, W_doc, W_uid, W_zip,
        W_pid,
    )
    return (lx.reshape(B, H, 144), nx.reshape(B, H, 144), y.reshape(B, 144),
            os_e, uid_e, zip_e)
